# baseline (speedup 1.0000x reference)
import functools

import jax
import jax.numpy as jnp
from jax.experimental import pallas as pl
from jax.experimental.pallas import tpu as pltpu

_IOU_THRESH = 0.5
_SCORE_THRESH = 0.05
_B = 512


def _nms_tiles_kernel(br8, x1r, y1r, x2r, y2r, keep_out, survr, *,
                      num_tiles):
    B = _B
    f32 = jnp.float32
    hi = jax.lax.Precision.HIGHEST

    survr[...] = jnp.zeros_like(survr)

    row_i = jax.lax.broadcasted_iota(jnp.int32, (B, B), 0)
    col_j = jax.lax.broadcasted_iota(jnp.int32, (B, B), 1)
    strict_lower = (row_i > col_j).astype(f32)
    lower_incl = (row_i >= col_j).astype(f32)
    eye = (row_i == col_j).astype(f32)
    iota_lane = jax.lax.broadcasted_iota(jnp.int32, (1, B), 1).astype(f32)

    def tile_body(t, count):
        x1 = x1r[pl.ds(t, 1), :]
        y1 = y1r[pl.ds(t, 1), :]
        x2 = x2r[pl.ds(t, 1), :]
        y2 = y2r[pl.ds(t, 1), :]
        area = (x2 - x1) * (y2 - y1)

        bt = br8[pl.ds(t * B, B), :]
        x1c, y1c = bt[:, 0:1], bt[:, 1:2]
        x2c, y2c = bt[:, 2:3], bt[:, 3:4]
        k0c = bt[:, 4:5]
        areac = (x2c - x1c) * (y2c - y1c)

        x1cb = jnp.broadcast_to(x1c, (B, B))
        y1cb = jnp.broadcast_to(y1c, (B, B))
        x2cb = jnp.broadcast_to(x2c, (B, B))
        y2cb = jnp.broadcast_to(y2c, (B, B))
        areacb = jnp.broadcast_to(areac, (B, B))

        nprev2 = (count + 2 * B - 1) // (2 * B)

        def blk_iou(q):
            sv = survr[:, pl.ds(pl.multiple_of(q * B, B), B)]
            sx1, sy1 = sv[0:1, :], sv[1:2, :]
            sx2, sy2 = sv[2:3, :], sv[3:4, :]
            sarea = (sx2 - sx1) * (sy2 - sy1)
            xx1 = jnp.maximum(x1cb, sx1)
            yy1 = jnp.maximum(y1cb, sy1)
            xx2 = jnp.minimum(x2cb, sx2)
            yy2 = jnp.minimum(y2cb, sy2)
            inter = jnp.maximum(xx2 - xx1, 0.0) * jnp.maximum(yy2 - yy1, 0.0)
            return inter / (areacb + sarea - inter + 1e-9)

        def cross_body(p, acc):
            iou0 = blk_iou(2 * p)
            iou1 = blk_iou(2 * p + 1)
            m = jnp.maximum(iou0, iou1)
            hf = jnp.maximum(m[:, :B // 2], m[:, B // 2:])
            return jnp.maximum(acc, hf)

        iou_acc = jax.lax.fori_loop(0, nprev2, cross_body,
                                    jnp.zeros((B, B // 2), f32))
        cross = jnp.max(iou_acc, axis=1, keepdims=True)
        k0m = jnp.where(cross > _IOU_THRESH, 0.0, k0c)

        xx1 = jnp.maximum(x1cb, x1)
        yy1 = jnp.maximum(y1cb, y1)
        xx2 = jnp.minimum(x2cb, x2)
        yy2 = jnp.minimum(y2cb, y2)
        inter = jnp.maximum(xx2 - xx1, 0.0) * jnp.maximum(yy2 - yy1, 0.0)
        iou = inter / (areacb + area - inter + 1e-9)
        mbt = (iou > _IOU_THRESH).astype(f32) * strict_lower

        def fp_cond(c):
            return c[1]

        def fp_body(c):
            k, _ = c
            supp = jnp.dot(mbt, k, preferred_element_type=f32)
            kn = jnp.where(supp > 0.5, 0.0, k0m)
            changed = jnp.sum(jnp.abs(kn - k)) > 0.0
            return kn, changed

        k, _ = jax.lax.while_loop(fp_cond, fp_body,
                                  (k0m, jnp.bool_(True)))

        incl = jnp.dot(lower_incl, k, preferred_element_type=f32)
        pos = incl - k
        onehot = ((pos == iota_lane) & (k > 0.5)).astype(f32)
        btt = jnp.concatenate([x1, y1, x2, y2], axis=0)
        compact = jnp.dot(btt, onehot, precision=hi,
                          preferred_element_type=f32)
        survr[0:4, pl.ds(pl.multiple_of(count, 128), B)] = compact

        k_row = jax.lax.dot_general(k, eye, (((0,), (0,)), ((), ())),
                                    preferred_element_type=f32)
        keep_out[pl.ds(t, 1), :] = k_row

        ksum = jnp.sum(k).astype(jnp.int32)
        return count + ((ksum + 127) // 128) * 128

    jax.lax.fori_loop(0, num_tiles, tile_body, jnp.int32(0))


def _nms_sorted(bp8, planes, num_tiles):
    nps = num_tiles * _B + 128 * num_tiles + 2 * _B
    nps = ((nps + 127) // 128) * 128
    fn = pl.pallas_call(
        functools.partial(_nms_tiles_kernel, num_tiles=num_tiles),
        out_shape=jax.ShapeDtypeStruct((num_tiles, _B), jnp.float32),
        scratch_shapes=[pltpu.VMEM((8, nps), jnp.float32)],
    )
    return fn(bp8, planes[0], planes[1], planes[2], planes[3])


def kernel(boxes, scores):
    n = boxes.shape[0]
    num_tiles = (n + _B - 1) // _B
    npad = num_tiles * _B

    order = jnp.argsort(-scores)
    b = jnp.take(boxes, order, axis=0)
    s = jnp.take(scores, order, axis=0)

    k0 = (s > _SCORE_THRESH).astype(jnp.float32)
    pad = npad - n
    bp = jnp.pad(b, ((0, pad), (0, 0)))
    k0p = jnp.pad(k0, (0, pad))
    bp8 = jnp.concatenate(
        [bp, k0p[:, None], jnp.zeros((npad, 3), jnp.float32)], axis=1)
    planes = bp.T.reshape(4, num_tiles, _B)

    keep_sorted = _nms_sorted(bp8, planes, num_tiles).reshape(-1)[:n]
    keep = jnp.zeros((n,), jnp.float32).at[order].set(keep_sorted)
    return jnp.concatenate(
        [boxes * keep[:, None], (scores * keep)[:, None]], axis=1)


# speedup vs baseline: 1.0129x; 1.0129x over previous
import functools

import jax
import jax.numpy as jnp
from jax.experimental import pallas as pl
from jax.experimental.pallas import tpu as pltpu

_IOU_THRESH = 0.5
_SCORE_THRESH = 0.05
_B = 256


def _nms_tiles_kernel(br8, x1r, y1r, x2r, y2r, keep_out, survr, *,
                      num_tiles):
    B = _B
    f32 = jnp.float32
    hi = jax.lax.Precision.HIGHEST

    survr[...] = jnp.zeros_like(survr)

    row_i = jax.lax.broadcasted_iota(jnp.int32, (B, B), 0)
    col_j = jax.lax.broadcasted_iota(jnp.int32, (B, B), 1)
    strict_lower = (row_i > col_j).astype(f32)
    lower_incl = (row_i >= col_j).astype(f32)
    eye = (row_i == col_j).astype(f32)
    iota_lane = jax.lax.broadcasted_iota(jnp.int32, (1, B), 1).astype(f32)

    def tile_body(t, count):
        x1 = x1r[pl.ds(t, 1), :]
        y1 = y1r[pl.ds(t, 1), :]
        x2 = x2r[pl.ds(t, 1), :]
        y2 = y2r[pl.ds(t, 1), :]
        area = (x2 - x1) * (y2 - y1)

        bt = br8[pl.ds(t * B, B), :]
        x1c, y1c = bt[:, 0:1], bt[:, 1:2]
        x2c, y2c = bt[:, 2:3], bt[:, 3:4]
        k0c = bt[:, 4:5]
        areac = (x2c - x1c) * (y2c - y1c)

        x1cb = jnp.broadcast_to(x1c, (B, B))
        y1cb = jnp.broadcast_to(y1c, (B, B))
        x2cb = jnp.broadcast_to(x2c, (B, B))
        y2cb = jnp.broadcast_to(y2c, (B, B))
        areacb = jnp.broadcast_to(areac, (B, B))

        nprev4 = (count + 4 * B - 1) // (4 * B)

        def blk_iou(q):
            sv = survr[:, pl.ds(pl.multiple_of(q * B, B), B)]
            sx1, sy1 = sv[0:1, :], sv[1:2, :]
            sx2, sy2 = sv[2:3, :], sv[3:4, :]
            sarea = sv[4:5, :]
            xx1 = jnp.maximum(x1cb, sx1)
            yy1 = jnp.maximum(y1cb, sy1)
            xx2 = jnp.minimum(x2cb, sx2)
            yy2 = jnp.minimum(y2cb, sy2)
            inter = jnp.maximum(xx2 - xx1, 0.0) * jnp.maximum(yy2 - yy1, 0.0)
            return inter / (areacb + sarea - inter + 1e-9)

        def cross_body(p, acc):
            m0 = jnp.maximum(blk_iou(4 * p), blk_iou(4 * p + 1))
            m1 = jnp.maximum(blk_iou(4 * p + 2), blk_iou(4 * p + 3))
            m = jnp.maximum(m0, m1)
            hf = jnp.maximum(m[:, :B // 2], m[:, B // 2:])
            return jnp.maximum(acc, hf)

        iou_acc = jax.lax.fori_loop(0, nprev4, cross_body,
                                    jnp.zeros((B, B // 2), f32))
        cross = jnp.max(iou_acc, axis=1, keepdims=True)
        k0m = jnp.where(cross > _IOU_THRESH, 0.0, k0c)

        xx1 = jnp.maximum(x1cb, x1)
        yy1 = jnp.maximum(y1cb, y1)
        xx2 = jnp.minimum(x2cb, x2)
        yy2 = jnp.minimum(y2cb, y2)
        inter = jnp.maximum(xx2 - xx1, 0.0) * jnp.maximum(yy2 - yy1, 0.0)
        iou = inter / (areacb + area - inter + 1e-9)
        mbt = (iou > _IOU_THRESH).astype(f32) * strict_lower

        def fp_cond(c):
            return c[1]

        def fp_body(c):
            k, _ = c
            supp = jnp.dot(mbt, k, preferred_element_type=f32)
            kn = jnp.where(supp > 0.5, 0.0, k0m)
            changed = jnp.sum(jnp.abs(kn - k)) > 0.0
            return kn, changed

        k, _ = jax.lax.while_loop(fp_cond, fp_body,
                                  (k0m, jnp.bool_(True)))

        incl = jnp.dot(lower_incl, k, preferred_element_type=f32)
        pos = incl - k
        onehot = ((pos == iota_lane) & (k > 0.5)).astype(f32)
        btt = jnp.concatenate([x1, y1, x2, y2, area], axis=0)
        compact = jnp.dot(btt, onehot, precision=hi,
                          preferred_element_type=f32)
        survr[0:5, pl.ds(pl.multiple_of(count, 128), B)] = compact

        k_row = jax.lax.dot_general(k, eye, (((0,), (0,)), ((), ())),
                                    preferred_element_type=f32)
        keep_out[pl.ds(t, 1), :] = k_row

        ksum = jnp.sum(k).astype(jnp.int32)
        return count + ((ksum + 127) // 128) * 128

    jax.lax.fori_loop(0, num_tiles, tile_body, jnp.int32(0))


def _nms_sorted(bp8, planes, num_tiles):
    nps = num_tiles * _B + 128 * num_tiles + 8 * _B
    nps = ((nps + 127) // 128) * 128
    fn = pl.pallas_call(
        functools.partial(_nms_tiles_kernel, num_tiles=num_tiles),
        out_shape=jax.ShapeDtypeStruct((num_tiles, _B), jnp.float32),
        scratch_shapes=[pltpu.VMEM((8, nps), jnp.float32)],
    )
    return fn(bp8, planes[0], planes[1], planes[2], planes[3])


def kernel(boxes, scores):
    n = boxes.shape[0]
    num_tiles = (n + _B - 1) // _B
    npad = num_tiles * _B

    order = jnp.argsort(-scores)
    b = jnp.take(boxes, order, axis=0)
    s = jnp.take(scores, order, axis=0)

    k0 = (s > _SCORE_THRESH).astype(jnp.float32)
    pad = npad - n
    bp = jnp.pad(b, ((0, pad), (0, 0)))
    k0p = jnp.pad(k0, (0, pad))
    bp8 = jnp.concatenate(
        [bp, k0p[:, None], jnp.zeros((npad, 3), jnp.float32)], axis=1)
    planes = bp.T.reshape(4, num_tiles, _B)

    keep_sorted = _nms_sorted(bp8, planes, num_tiles).reshape(-1)[:n]
    keep = jnp.zeros((n,), jnp.float32).at[order].set(keep_sorted)
    return jnp.concatenate(
        [boxes * keep[:, None], (scores * keep)[:, None]], axis=1)


# speedup vs baseline: 1.2092x; 1.1939x over previous
import functools

import jax
import jax.numpy as jnp
from jax.experimental import pallas as pl
from jax.experimental.pallas import tpu as pltpu

_IOU_THRESH = 0.5
_SCORE_THRESH = 0.05
_B = 256


def _nms_tiles_kernel(br8, x1r, y1r, x2r, y2r, keep_out, survr, *,
                      num_tiles):
    B = _B
    f32 = jnp.float32
    hi = jax.lax.Precision.HIGHEST

    survr[...] = jnp.zeros_like(survr)

    row_i = jax.lax.broadcasted_iota(jnp.int32, (B, B), 0)
    col_j = jax.lax.broadcasted_iota(jnp.int32, (B, B), 1)
    strict_lower = (row_i > col_j).astype(f32)
    lower_incl = (row_i >= col_j).astype(f32)
    eye = (row_i == col_j).astype(f32)
    iota_lane = jax.lax.broadcasted_iota(jnp.int32, (1, B), 1).astype(f32)

    def tile_body(t, count):
        x1 = x1r[pl.ds(t, 1), :]
        y1 = y1r[pl.ds(t, 1), :]
        x2 = x2r[pl.ds(t, 1), :]
        y2 = y2r[pl.ds(t, 1), :]
        area = (x2 - x1) * (y2 - y1)

        bt = br8[pl.ds(t * B, B), :]
        x1c, y1c = bt[:, 0:1], bt[:, 1:2]
        x2c, y2c = bt[:, 2:3], bt[:, 3:4]
        k0c = bt[:, 4:5]
        areac = (x2c - x1c) * (y2c - y1c)

        x1cb = jnp.broadcast_to(x1c, (B, B))
        y1cb = jnp.broadcast_to(y1c, (B, B))
        x2cb = jnp.broadcast_to(x2c, (B, B))
        y2cb = jnp.broadcast_to(y2c, (B, B))
        areacb = jnp.broadcast_to(areac, (B, B))

        nprev2 = (count + 2 * B - 1) // (2 * B)

        def blk_iou(q):
            sv = survr[:, pl.ds(pl.multiple_of(q * B, B), B)]
            sx1, sy1 = sv[0:1, :], sv[1:2, :]
            sx2, sy2 = sv[2:3, :], sv[3:4, :]
            sarea = (sx2 - sx1) * (sy2 - sy1)
            xx1 = jnp.maximum(x1cb, sx1)
            yy1 = jnp.maximum(y1cb, sy1)
            xx2 = jnp.minimum(x2cb, sx2)
            yy2 = jnp.minimum(y2cb, sy2)
            inter = jnp.maximum(xx2 - xx1, 0.0) * jnp.maximum(yy2 - yy1, 0.0)
            return inter / (areacb + sarea - inter + 1e-9)

        def cross_body(p, acc):
            iou0 = blk_iou(2 * p)
            iou1 = blk_iou(2 * p + 1)
            m = jnp.maximum(iou0, iou1)
            hf = jnp.maximum(m[:, :B // 2], m[:, B // 2:])
            return jnp.maximum(acc, hf)

        iou_acc = jax.lax.fori_loop(0, nprev2, cross_body,
                                    jnp.zeros((B, B // 2), f32))
        cross = jnp.max(iou_acc, axis=1, keepdims=True)
        k0m = jnp.where(cross > _IOU_THRESH, 0.0, k0c)

        xx1 = jnp.maximum(x1cb, x1)
        yy1 = jnp.maximum(y1cb, y1)
        xx2 = jnp.minimum(x2cb, x2)
        yy2 = jnp.minimum(y2cb, y2)
        inter = jnp.maximum(xx2 - xx1, 0.0) * jnp.maximum(yy2 - yy1, 0.0)
        iou = inter / (areacb + area - inter + 1e-9)
        mbt = (iou > _IOU_THRESH).astype(f32) * strict_lower

        def fp_cond(c):
            return c[1]

        def fp_body(c):
            k, _ = c
            supp = jnp.dot(mbt, k, preferred_element_type=f32)
            kn = jnp.where(supp > 0.5, 0.0, k0m)
            changed = jnp.sum(jnp.abs(kn - k)) > 0.0
            return kn, changed

        k, _ = jax.lax.while_loop(fp_cond, fp_body,
                                  (k0m, jnp.bool_(True)))

        incl = jnp.dot(lower_incl, k, preferred_element_type=f32)
        pos = incl - k
        onehot = ((pos == iota_lane) & (k > 0.5)).astype(f32)
        btt = jnp.concatenate([x1, y1, x2, y2], axis=0)
        compact = jnp.dot(btt, onehot, precision=hi,
                          preferred_element_type=f32)
        survr[0:4, pl.ds(pl.multiple_of(count, 128), B)] = compact

        k_row = jax.lax.dot_general(k, eye, (((0,), (0,)), ((), ())),
                                    preferred_element_type=f32)
        keep_out[pl.ds(t, 1), :] = k_row

        ksum = jnp.sum(k).astype(jnp.int32)
        return count + ((ksum + 127) // 128) * 128

    jax.lax.fori_loop(0, num_tiles, tile_body, jnp.int32(0))


def _nms_sorted(bp8, planes, num_tiles):
    nps = num_tiles * _B + 128 * num_tiles + 8 * _B
    nps = ((nps + 127) // 128) * 128
    fn = pl.pallas_call(
        functools.partial(_nms_tiles_kernel, num_tiles=num_tiles),
        out_shape=jax.ShapeDtypeStruct((num_tiles, _B), jnp.float32),
        scratch_shapes=[pltpu.VMEM((8, nps), jnp.float32)],
    )
    return fn(bp8, planes[0], planes[1], planes[2], planes[3])


def kernel(boxes, scores):
    n = boxes.shape[0]
    num_tiles = (n + _B - 1) // _B
    npad = num_tiles * _B

    iota = jnp.arange(n, dtype=jnp.int32)
    negs, sx1, sy1, sx2, sy2, order = jax.lax.sort(
        (-scores, boxes[:, 0], boxes[:, 1], boxes[:, 2], boxes[:, 3],
         iota), num_keys=1, is_stable=True)
    s = -negs

    k0 = (s > _SCORE_THRESH).astype(jnp.float32)
    pad = npad - n
    k0p = jnp.pad(k0, (0, pad))
    cols = [jnp.pad(c, (0, pad)) for c in (sx1, sy1, sx2, sy2)]
    bp8 = jnp.concatenate(
        [jnp.stack(cols, axis=1), k0p[:, None],
         jnp.zeros((npad, 3), jnp.float32)], axis=1)
    planes = jnp.stack(cols, axis=0).reshape(4, num_tiles, _B)

    keep_sorted = _nms_sorted(bp8, planes, num_tiles).reshape(-1)[:n]
    keep = jnp.zeros((n,), jnp.float32).at[order].set(keep_sorted)
    return jnp.concatenate(
        [boxes * keep[:, None], (scores * keep)[:, None]], axis=1)
